# baseline (speedup 1.0000x reference)
import numpy as np

import jax
import jax.numpy as jnp
from jax.experimental import pallas as pl

B, N, IN_DIM = 8, 512, 64
HEADS, HEAD_DIM = 4, 16
OUT_DIM = HEADS * HEAD_DIM
AGG_W = OUT_DIM + HEADS
BPS = 2

_S68 = np.zeros((AGG_W, OUT_DIM), dtype=np.float32)
for _h in range(HEADS):
    _S68[OUT_DIM + _h, _h * HEAD_DIM:(_h + 1) * HEAD_DIM] = 1.0

_EYE = np.where(np.eye(N, dtype=bool), 0.0, -1e30).astype(jnp.bfloat16)


def _gat_one(xb, adjb, eyeb, w, wsrc, wdst, s68):
    h = jnp.dot(xb, w, preferred_element_type=jnp.float32)

    has_edge = jnp.max(adjb) > 0.5
    bias = jnp.where(adjb > 0.5, 0.0, -1e30).astype(jnp.bfloat16)
    bias = jnp.where(has_edge, bias, eyeb)

    es = jnp.dot(xb, wsrc, preferred_element_type=jnp.float32)
    edT = jax.lax.dot_general(wdst, xb, (((0,), (1,)), ((), ())),
                              preferred_element_type=jnp.float32)
    Ms = jnp.max(es, axis=0, keepdims=True)
    log2e = float(np.log2(np.e))
    sa = (es - Ms) * log2e
    sa_b = sa.astype(jnp.bfloat16)
    sb_b = (0.2 * sa).astype(jnp.bfloat16)

    ones4 = jnp.full((N, HEADS), 1.0, dtype=jnp.float32)
    rhs = jnp.concatenate([h, ones4], axis=1).astype(jnp.bfloat16)

    lane = jax.lax.broadcasted_iota(jnp.int32, (N, AGG_W), 1)
    total = jnp.zeros((N, AGG_W), dtype=jnp.float32)
    for hh in range(HEADS):
        s = sa_b[:, hh:hh + 1]
        s2 = sb_b[:, hh:hh + 1]
        mm = Ms[:, hh:hh + 1] + edT[hh:hh + 1, :]
        mrow = jnp.maximum(mm, 0.2 * mm)
        d1 = ((mm - mrow) * log2e).astype(jnp.bfloat16)
        d2 = ((0.2 * mm - mrow) * log2e).astype(jnp.bfloat16)
        t = jnp.maximum(s + d1, s2 + d2) + bias
        exb = jnp.exp2(t)
        agg = jax.lax.dot_general(exb, rhs, (((0,), (0,)), ((), ())),
                                  preferred_element_type=jnp.float32)
        head_lanes = ((lane >= hh * HEAD_DIM) & (lane < (hh + 1) * HEAD_DIM)
                      ) | (lane == OUT_DIM + hh)
        total = total + jnp.where(head_lanes, agg, 0.0)

    den_guarded = jnp.where(lane >= OUT_DIM, total, 1.0)
    recip = 1.0 / (den_guarded + 1e-16)
    scale = jnp.dot(recip, s68, preferred_element_type=jnp.float32)
    return total[:, :OUT_DIM] * scale


def _gat_kernel(x_ref, adj_ref, eye_ref, w_ref, wsrc_ref, wdst_ref, s68_ref,
                out_ref):
    for bb in range(BPS):
        out_ref[bb] = _gat_one(x_ref[bb], adj_ref[bb], eye_ref[...],
                               w_ref[...], wsrc_ref[...], wdst_ref[...],
                               s68_ref[...])


@jax.jit
def kernel(x, adj, W, a_src, a_dst):
    w_flat = W.reshape(IN_DIM, OUT_DIM)
    wsrc = jnp.einsum('dhf,hf->dh', W, a_src)
    wdst = jnp.einsum('dhf,hf->dh', W, a_dst)
    s68 = jnp.asarray(_S68)
    eyeb = jnp.asarray(_EYE)
    return pl.pallas_call(
        _gat_kernel,
        grid=(B // BPS,),
        in_specs=[
            pl.BlockSpec((BPS, N, IN_DIM), lambda b: (b, 0, 0)),
            pl.BlockSpec((BPS, N, N), lambda b: (b, 0, 0)),
            pl.BlockSpec((N, N), lambda b: (0, 0)),
            pl.BlockSpec((IN_DIM, OUT_DIM), lambda b: (0, 0)),
            pl.BlockSpec((IN_DIM, HEADS), lambda b: (0, 0)),
            pl.BlockSpec((IN_DIM, HEADS), lambda b: (0, 0)),
            pl.BlockSpec((AGG_W, OUT_DIM), lambda b: (0, 0)),
        ],
        out_specs=pl.BlockSpec((BPS, N, OUT_DIM), lambda b: (b, 0, 0)),
        out_shape=jax.ShapeDtypeStruct((B, N, OUT_DIM), jnp.float32),
    )(x, adj, eyeb, w_flat, wsrc, wdst, s68)


# speedup vs baseline: 1.0159x; 1.0159x over previous
import numpy as np

import jax
import jax.numpy as jnp
from jax.experimental import pallas as pl

B, N, IN_DIM = 8, 512, 64
HEADS, HEAD_DIM = 4, 16
OUT_DIM = HEADS * HEAD_DIM
AGG_W = OUT_DIM + HEADS
BPS = 2

_S68 = np.zeros((AGG_W, OUT_DIM), dtype=np.float32)
for _h in range(HEADS):
    _S68[OUT_DIM + _h, _h * HEAD_DIM:(_h + 1) * HEAD_DIM] = 1.0

_EYE = np.eye(N, dtype=np.float32).astype(jnp.bfloat16)


def _gat_one(xb, adjb, eyeb, w, wsrc, wdst, s68):
    h = jnp.dot(xb, w, preferred_element_type=jnp.float32)

    has_edge = jnp.max(adjb) > 0.5
    b01 = jnp.where(adjb > 0.5, 1.0, 0.0).astype(jnp.bfloat16)
    b01 = jnp.where(has_edge, b01, eyeb)

    es = jnp.dot(xb, wsrc, preferred_element_type=jnp.float32)
    edT = jax.lax.dot_general(wdst, xb, (((0,), (1,)), ((), ())),
                              preferred_element_type=jnp.float32)
    Ms = jnp.max(es, axis=0, keepdims=True)
    log2e = float(np.log2(np.e))
    sa = (es - Ms) * log2e
    sa_b = sa.astype(jnp.bfloat16)
    sb_b = (0.2 * sa).astype(jnp.bfloat16)

    ones4 = jnp.full((N, HEADS), 1.0, dtype=jnp.float32)
    rhs = jnp.concatenate([h, ones4], axis=1).astype(jnp.bfloat16)

    lane = jax.lax.broadcasted_iota(jnp.int32, (N, AGG_W), 1)
    total = jnp.zeros((N, AGG_W), dtype=jnp.float32)
    ts, exbs = [], []
    for hh in range(HEADS):
        s = sa_b[:, hh:hh + 1]
        s2 = sb_b[:, hh:hh + 1]
        mm = Ms[:, hh:hh + 1] + edT[hh:hh + 1, :]
        mrow = jnp.maximum(mm, 0.2 * mm)
        d1 = ((mm - mrow) * log2e).astype(jnp.bfloat16)
        d2 = ((0.2 * mm - mrow) * log2e).astype(jnp.bfloat16)
        ts.append(jnp.maximum(s + d1, s2 + d2))
    for hh in range(HEADS):
        exbs.append(jnp.exp2(ts[hh]) * b01)
    for hh in range(HEADS):
        agg = jax.lax.dot_general(exbs[hh], rhs, (((0,), (0,)), ((), ())),
                                  preferred_element_type=jnp.float32)
        head_lanes = ((lane >= hh * HEAD_DIM) & (lane < (hh + 1) * HEAD_DIM)
                      ) | (lane == OUT_DIM + hh)
        total = total + jnp.where(head_lanes, agg, 0.0)

    den_guarded = jnp.where(lane >= OUT_DIM, total, 1.0)
    recip = 1.0 / (den_guarded + 1e-16)
    scale = jnp.dot(recip, s68, preferred_element_type=jnp.float32)
    return total[:, :OUT_DIM] * scale


def _gat_kernel(x_ref, adj_ref, eye_ref, w_ref, wsrc_ref, wdst_ref, s68_ref,
                out_ref):
    for bb in range(BPS):
        out_ref[bb] = _gat_one(x_ref[bb], adj_ref[bb], eye_ref[...],
                               w_ref[...], wsrc_ref[...], wdst_ref[...],
                               s68_ref[...])


@jax.jit
def kernel(x, adj, W, a_src, a_dst):
    w_flat = W.reshape(IN_DIM, OUT_DIM)
    wsrc = jnp.einsum('dhf,hf->dh', W, a_src)
    wdst = jnp.einsum('dhf,hf->dh', W, a_dst)
    s68 = jnp.asarray(_S68)
    eyeb = jnp.asarray(_EYE)
    return pl.pallas_call(
        _gat_kernel,
        grid=(B // BPS,),
        in_specs=[
            pl.BlockSpec((BPS, N, IN_DIM), lambda b: (b, 0, 0)),
            pl.BlockSpec((BPS, N, N), lambda b: (b, 0, 0)),
            pl.BlockSpec((N, N), lambda b: (0, 0)),
            pl.BlockSpec((IN_DIM, OUT_DIM), lambda b: (0, 0)),
            pl.BlockSpec((IN_DIM, HEADS), lambda b: (0, 0)),
            pl.BlockSpec((IN_DIM, HEADS), lambda b: (0, 0)),
            pl.BlockSpec((AGG_W, OUT_DIM), lambda b: (0, 0)),
        ],
        out_specs=pl.BlockSpec((BPS, N, OUT_DIM), lambda b: (b, 0, 0)),
        out_shape=jax.ShapeDtypeStruct((B, N, OUT_DIM), jnp.float32),
    )(x, adj, eyeb, w_flat, wsrc, wdst, s68)
